# baseline (speedup 1.0000x reference)
import functools

import jax
import jax.numpy as jnp
from jax import lax
from jax.experimental import pallas as pl
from jax.experimental.pallas import tpu as pltpu
from jax.experimental.pallas import tpu_sc as plsc

B, K, L = 32, 1024, 576
LK = L * K
RPC = 32
NCHUNK = L // RPC
CH = RPC * K
NSCAT = 5

_mesh = plsc.VectorSubcoreMesh(core_axis_name="c", subcore_axis_name="s")


@functools.partial(
    pl.kernel,
    out_type=jax.ShapeDtypeStruct((B * LK,), jnp.float32),
    mesh=_mesh,
    scratch_types=[
        pltpu.VMEM((RPC, K), jnp.float32),
        pltpu.VMEM((RPC, K), jnp.float32),
        pltpu.VMEM((CH,), jnp.float32),
        pltpu.VMEM((NSCAT, 128), jnp.int32),
        pltpu.VMEM((128,), jnp.float32),
        pltpu.SemaphoreType.DMA,
        pltpu.SemaphoreType.DMA,
        pltpu.SemaphoreType.DMA,
        pltpu.SemaphoreType.DMA,
    ],
    compiler_params=pltpu.CompilerParams(needs_layout_passes=False),
)
def _argmax_onehot(x_hbm, out_hbm, buf0, buf1, zbuf, fidx_ref, ones_ref,
                   si0, si1, semz, semsc):
    b = lax.axis_index("s") * 2 + lax.axis_index("c")
    base = b * LK
    bufs = (buf0, buf1)
    in_sems = (si0, si1)

    h_in = [
        pltpu.async_copy(x_hbm.at[b, pl.ds(c * RPC, RPC), :], bufs[c],
                         in_sems[c])
        for c in range(2)
    ]

    zv = jnp.zeros((16,), jnp.float32)

    def zset(i, _):
        zbuf[pl.ds(i * 16, 16)] = zv
        return 0

    lax.fori_loop(0, CH // 16, zset, 0, unroll=8)

    onesv = jnp.full((16,), 1.0, jnp.float32)
    for t in range(8):
        ones_ref[pl.ds(t * 16, 16)] = onesv

    iota = lax.iota(jnp.int32, 16)
    ninf = jnp.full((16,), -jnp.inf, jnp.float32)
    izero = jnp.zeros((16,), jnp.int32)
    big = jnp.full((16,), jnp.int32(1 << 30), jnp.int32)
    zhandles = []

    for c in range(NCHUNK):
        zhandles.append(pltpu.async_copy(
            zbuf, out_hbm.at[pl.ds(base + c * CH, CH)], semz))

        h_in[c % 2].wait()
        buf = bufs[c % 2]

        def pbody(p, carry, buf=buf):
            acc0, acc1 = carry

            def jbody(j, c2, buf=buf, p=p):
                m0, ci0, m1, ci1, jv = c2
                v0 = buf[p, pl.ds(j * 16, 16)]
                v1 = buf[p + 16, pl.ds(j * 16, 16)]
                g0 = v0 > m0
                g1 = v1 > m1
                return (
                    jnp.where(g0, v0, m0), jnp.where(g0, jv, ci0),
                    jnp.where(g1, v1, m1), jnp.where(g1, jv, ci1),
                    jv + 1,
                )

            m0, ci0, m1, ci1, _ = lax.fori_loop(
                0, K // 16, jbody, (ninf, izero, ninf, izero, izero),
                unroll=8)

            def tail(m, ci):
                hm = jnp.max(m)
                cand = jnp.where(m == hm, ci * 16 + iota, big)
                return jnp.min(cand)

            lm = iota == p
            acc0 = jnp.where(lm, tail(m0, ci0), acc0)
            acc1 = jnp.where(lm, tail(m1, ci1), acc1)
            return (acc0, acc1)

        i0, i1 = lax.fori_loop(0, 16, pbody, (izero, izero))

        if c + 2 < NCHUNK:
            h_in[c % 2] = pltpu.async_copy(
                x_hbm.at[b, pl.ds((c + 2) * RPC, RPC), :], bufs[c % 2],
                in_sems[c % 2])

        e0 = c * RPC
        f0 = base + (e0 + iota) * K + i0
        f1 = base + (e0 + 16 + iota) * K + i1
        fidx_ref[e0 // 128, pl.ds(e0 % 128, 16)] = f0
        fidx_ref[e0 // 128, pl.ds(e0 % 128 + 16, 16)] = f1

    last = fidx_ref[4, pl.ds(48, 16)]
    dup = jnp.full((16,), last[15], jnp.int32)
    for t in range(4):
        fidx_ref[4, pl.ds(64 + t * 16, 16)] = dup

    for h in zhandles:
        h.wait()

    shandles = [
        pltpu.async_copy(ones_ref, out_hbm.at[fidx_ref.at[j]], semsc)
        for j in range(NSCAT)
    ]
    for h in shandles:
        h.wait()


def kernel(x):
    xt = jnp.transpose(x, (0, 2, 1))
    flat = _argmax_onehot(xt)
    return flat.reshape(B, L, K).transpose(0, 2, 1)


# speedup vs baseline: 2.5206x; 2.5206x over previous
import functools

import jax
import jax.numpy as jnp
from jax import lax
from jax.experimental import pallas as pl
from jax.experimental.pallas import tpu as pltpu
from jax.experimental.pallas import tpu_sc as plsc

B, K, L = 32, 1024, 576
RPC = 32
NCHUNK = L // RPC

_mesh = plsc.VectorSubcoreMesh(core_axis_name="c", subcore_axis_name="s")


@functools.partial(
    pl.kernel,
    out_type=jax.ShapeDtypeStruct((B, L, K), jnp.float32),
    mesh=_mesh,
    scratch_types=[
        pltpu.VMEM((RPC, K), jnp.float32),
        pltpu.VMEM((RPC, K), jnp.float32),
        pltpu.VMEM((RPC, K), jnp.float32),
        pltpu.SemaphoreType.DMA,
        pltpu.SemaphoreType.DMA,
        pltpu.SemaphoreType.DMA,
    ],
    compiler_params=pltpu.CompilerParams(needs_layout_passes=False),
)
def _argmax_onehot(x_hbm, out_hbm, buf0, buf1, obuf, si0, si1, so):
    b = lax.axis_index("s") * 2 + lax.axis_index("c")
    bufs = (buf0, buf1)
    in_sems = (si0, si1)

    h_in = [
        pltpu.async_copy(x_hbm.at[b, pl.ds(c * RPC, RPC), :], bufs[c],
                         in_sems[c])
        for c in range(2)
    ]

    zv = jnp.zeros((16,), jnp.float32)

    def zbody(i, _):
        def inner(j, _, i=i):
            obuf[i, pl.ds(j * 16, 16)] = zv
            return 0
        lax.fori_loop(0, K // 16, inner, 0, unroll=8)
        return 0

    lax.fori_loop(0, RPC, zbody, 0)

    iota = lax.iota(jnp.int32, 16)
    lanes0 = iota
    lanes1 = iota + 16
    onev = jnp.full((16,), 1.0, jnp.float32)
    ninf = jnp.full((16,), -jnp.inf, jnp.float32)
    izero = jnp.zeros((16,), jnp.int32)
    big = jnp.full((16,), jnp.int32(1 << 30), jnp.int32)
    h_out = None
    prev = None

    for c in range(NCHUNK):
        h_in[c % 2].wait()
        buf = bufs[c % 2]

        def pbody(p, carry, buf=buf):
            acc0, acc1 = carry

            def jbody(j, c2, buf=buf, p=p):
                m0, ci0, m1, ci1, jv = c2
                v0 = buf[p, pl.ds(j * 16, 16)]
                v1 = buf[p + 16, pl.ds(j * 16, 16)]
                g0 = v0 > m0
                g1 = v1 > m1
                return (
                    jnp.where(g0, v0, m0), jnp.where(g0, jv, ci0),
                    jnp.where(g1, v1, m1), jnp.where(g1, jv, ci1),
                    jv + 1,
                )

            m0, ci0, m1, ci1, _ = lax.fori_loop(
                0, K // 16, jbody, (ninf, izero, ninf, izero, izero),
                unroll=8)

            def tail(m, ci):
                hm = jnp.max(m)
                cand = jnp.where(m == hm, ci * 16 + iota, big)
                return jnp.min(cand)

            lm = iota == p
            acc0 = jnp.where(lm, tail(m0, ci0), acc0)
            acc1 = jnp.where(lm, tail(m1, ci1), acc1)
            return (acc0, acc1)

        i0, i1 = lax.fori_loop(0, 16, pbody, (izero, izero))

        if c + 2 < NCHUNK:
            h_in[c % 2] = pltpu.async_copy(
                x_hbm.at[b, pl.ds((c + 2) * RPC, RPC), :], bufs[c % 2],
                in_sems[c % 2])

        if h_out is not None:
            h_out.wait()
            pi0, pi1 = prev
            plsc.store_scatter(obuf, [lanes0, pi0], zv)
            plsc.store_scatter(obuf, [lanes1, pi1], zv)
        plsc.store_scatter(obuf, [lanes0, i0], onev)
        plsc.store_scatter(obuf, [lanes1, i1], onev)
        prev = (i0, i1)

        h_out = pltpu.async_copy(
            obuf, out_hbm.at[b, pl.ds(c * RPC, RPC), :], so)

    h_out.wait()


def kernel(x):
    xt = jnp.transpose(x, (0, 2, 1))
    ot = _argmax_onehot(xt)
    return jnp.transpose(ot, (0, 2, 1))
